# Initial kernel scaffold; baseline (speedup 1.0000x reference)
import jax
import jax.numpy as jnp
from jax.experimental import pallas as pl


def kernel(edge_vectors, distances, cutoffs, node_species, senders, receivers, params):
    raise NotImplementedError("write your pallas kernel here")


# speedup vs baseline: 9.7227x; 9.7227x over previous
import functools

import jax
import jax.numpy as jnp
import numpy as np
from jax import lax
from jax.experimental import pallas as pl
from jax.experimental.pallas import tpu as pltpu
from jax.experimental.pallas import tpu_sc as plsc

_N = 10000
_E = 320000
_C = 128
_H = 4
_DH = _C // _H
_DG = 16
_GF = 32
_NRBF = 32
_NSP = 100
_CUTOFF = 5.0
_AVG_NN = 32.0
_SPHC_NORM = 32.0

_BE = 2560
_GE = _E // _BE
_BN = 2000
_GN = _N // _BN

_RD = 176
_SD = 304

_f32 = jnp.float32


def _sel(nin, nout, group):
    d = lax.broadcasted_iota(jnp.int32, (nin, nout), 0) // group
    h = lax.broadcasted_iota(jnp.int32, (nin, nout), 1)
    return jnp.where(d == h, 1.0, 0.0).astype(_f32)


def _selT(nin, nout, group):
    h = lax.broadcasted_iota(jnp.int32, (nin, nout), 0)
    d = lax.broadcasted_iota(jnp.int32, (nin, nout), 1) // group
    return jnp.where(d == h, 1.0, 0.0).astype(_f32)


def _rep28():
    i = lax.broadcasted_iota(jnp.int32, (2, 8), 0)
    j = lax.broadcasted_iota(jnp.int32, (2, 8), 1)
    return jnp.where((j < 3) == (i == 0), 1.0, 0.0).astype(_f32)


def _irrep_norm2(a):
    return jnp.concatenate(
        [jnp.sum(a[:, :3] * a[:, :3], axis=-1, keepdims=True),
         jnp.sum(a[:, 3:] * a[:, 3:], axis=-1, keepdims=True)], axis=-1)


def _dot(a, b):
    return jnp.dot(a, b, preferred_element_type=_f32)


def _edge_pre_body(ev, d, cut, ef_o, sh_o, shc_o):
    mu0 = np.float32(np.exp(-_CUTOFF))
    dmu = np.float32((1.0 - np.exp(-_CUTOFF)) / (_NRBF - 1))
    mus = mu0 + dmu * lax.broadcasted_iota(jnp.int32, (1, _NRBF), 1).astype(_f32)
    beta = np.float32((2.0 / _NRBF * (1.0 - np.exp(-_CUTOFF))) ** -2)
    d1 = d[...]
    ef_o[...] = jnp.exp(-beta * (jnp.exp(-d1) - mus) ** 2)
    v = ev[...]
    r = jnp.sqrt(jnp.sum(v * v, axis=-1, keepdims=True) + 1e-12)
    u = v / r
    x = u[:, 0:1]
    y = u[:, 1:2]
    z = u[:, 2:3]
    s3 = np.float32(3.0 ** 0.5)
    s15 = np.float32(15.0 ** 0.5)
    s5 = np.float32(5.0 ** 0.5)
    sh = jnp.concatenate(
        [s3 * x, s3 * y, s3 * z,
         s15 * x * y, s15 * y * z, 0.5 * s5 * (3.0 * z * z - 1.0),
         s15 * x * z, 0.5 * s15 * (x * x - y * y)], axis=-1)
    sh_o[...] = sh
    shc_o[...] = sh * cut[...]


_edge_pre = pl.pallas_call(
    _edge_pre_body,
    grid=(_GE,),
    in_specs=[pl.BlockSpec((_BE, 3), lambda i: (i, 0)),
              pl.BlockSpec((_BE, 1), lambda i: (i, 0)),
              pl.BlockSpec((_BE, 1), lambda i: (i, 0))],
    out_specs=[pl.BlockSpec((_BE, _NRBF), lambda i: (i, 0)),
               pl.BlockSpec((_BE, 8), lambda i: (i, 0)),
               pl.BlockSpec((_BE, 8), lambda i: (i, 0))],
    out_shape=[jax.ShapeDtypeStruct((_E, _NRBF), _f32),
               jax.ShapeDtypeStruct((_E, 8), _f32),
               jax.ShapeDtypeStruct((_E, 8), _f32)],
)


def _node_init_body(sp, chi0, chi1, emb, h_o, chi_o):
    s = sp[...]
    ids = lax.broadcasted_iota(jnp.int32, (1, _NSP), 1)
    oh = (s == ids).astype(_f32)
    h_o[...] = _dot(oh, emb[...])
    chi_o[...] = (chi0[0] + chi1[0]) * np.float32(1.0 / _SPHC_NORM)


_node_init = pl.pallas_call(
    _node_init_body,
    grid=(_GN,),
    in_specs=[pl.BlockSpec((_BN, 1), lambda i: (i, 0)),
              pl.BlockSpec((1, _BN, 8), lambda i: (0, i, 0)),
              pl.BlockSpec((1, _BN, 8), lambda i: (1, i, 0)),
              pl.BlockSpec((_NSP, _C), lambda i: (0, 0))],
    out_specs=[pl.BlockSpec((_BN, _C), lambda i: (i, 0)),
               pl.BlockSpec((_BN, 8), lambda i: (i, 0))],
    out_shape=[jax.ShapeDtypeStruct((_N, _C), _f32),
               jax.ShapeDtypeStruct((_N, 8), _f32)],
)


def _pack_qk_body(h, chi, wq, wk, wqg, wkg, r_o, s_o):
    hh = h[...]
    chiv = chi[...]
    z = jnp.zeros((hh.shape[0], 8), _f32)
    r_o[...] = jnp.concatenate(
        [_dot(hh, wq[...]), _dot(hh, wqg[...]), chiv, z], axis=-1)
    s_o[...] = jnp.concatenate(
        [_dot(hh, wk[...]), hh, _dot(hh, wkg[...]), chiv, z], axis=-1)


_pack_qk = pl.pallas_call(
    _pack_qk_body,
    grid=(_GN,),
    in_specs=[pl.BlockSpec((_BN, _C), lambda i: (i, 0)),
              pl.BlockSpec((_BN, 8), lambda i: (i, 0)),
              pl.BlockSpec((_C, _C), lambda i: (0, 0)),
              pl.BlockSpec((_C, _C), lambda i: (0, 0)),
              pl.BlockSpec((_C, _GF), lambda i: (0, 0)),
              pl.BlockSpec((_C, _GF), lambda i: (0, 0))],
    out_specs=[pl.BlockSpec((_BN, _RD), lambda i: (i, 0)),
               pl.BlockSpec((_BN, _SD), lambda i: (i, 0))],
    out_shape=[jax.ShapeDtypeStruct((_N, _RD), _f32),
               jax.ShapeDtypeStruct((_N, _SD), _f32)],
)


def _edge_compute_body(ef, rg, sg, sh, cut,
                       fbrW1, fbrb1, fbrW2, fbrb2,
                       fbsW1, fbsb1, fbsW2, fbsb2,
                       gbrW1, gbrb1, gbrW2, gbrb2,
                       gbsW1, gbsb1, gbsW2, gbsb2,
                       msg_o, mchi_o):
    silu = jax.nn.silu
    rgv = rg[...]
    sgv = sg[...]
    qr = rgv[:, :_C]
    qgr = rgv[:, _C:_C + _GF]
    chir = rgv[:, _C + _GF:_C + _GF + 8]
    ks = sgv[:, :_C]
    vs = sgv[:, _C:2 * _C]
    kgs = sgv[:, 2 * _C:2 * _C + _GF]
    chis = sgv[:, 2 * _C + _GF:2 * _C + _GF + 8]
    chij = chis - chir
    cs = _irrep_norm2(chij)
    efv = ef[...]
    cutv = cut[...]
    w = (_dot(silu(_dot(efv, fbrW1[...]) + fbrb1[...]), fbrW2[...]) + fbrb2[...]
         + _dot(silu(_dot(cs, fbsW1[...]) + fbsb1[...]), fbsW2[...]) + fbsb2[...])
    p = qr * w * ks
    alpha = _dot(p, _sel(_C, _H, _DH)) * np.float32(_DH ** -0.5) * cutv
    msg_o[...] = _dot(alpha, _selT(_H, _C, _DH)) * vs
    wg = (_dot(silu(_dot(efv, gbrW1[...]) + gbrb1[...]), gbrW2[...]) + gbrb2[...]
          + _dot(silu(_dot(cs, gbsW1[...]) + gbsb1[...]), gbsW2[...]) + gbsb2[...])
    pg = qgr * wg * kgs
    ag = _dot(pg, _sel(_GF, 2, _DG)) * np.float32(_DG ** -0.5) * cutv
    mchi_o[...] = _dot(ag, _rep28()) * sh[...]


_edge_compute = pl.pallas_call(
    _edge_compute_body,
    grid=(_GE,),
    in_specs=[pl.BlockSpec((_BE, _NRBF), lambda i: (i, 0)),
              pl.BlockSpec((_BE, _RD), lambda i: (i, 0)),
              pl.BlockSpec((_BE, _SD), lambda i: (i, 0)),
              pl.BlockSpec((_BE, 8), lambda i: (i, 0)),
              pl.BlockSpec((_BE, 1), lambda i: (i, 0)),
              pl.BlockSpec((_NRBF, 128), lambda i: (0, 0)),
              pl.BlockSpec((1, 128), lambda i: (0, 0)),
              pl.BlockSpec((128, _C), lambda i: (0, 0)),
              pl.BlockSpec((1, _C), lambda i: (0, 0)),
              pl.BlockSpec((2, 128), lambda i: (0, 0)),
              pl.BlockSpec((1, 128), lambda i: (0, 0)),
              pl.BlockSpec((128, _C), lambda i: (0, 0)),
              pl.BlockSpec((1, _C), lambda i: (0, 0)),
              pl.BlockSpec((_NRBF, 64), lambda i: (0, 0)),
              pl.BlockSpec((1, 64), lambda i: (0, 0)),
              pl.BlockSpec((64, _GF), lambda i: (0, 0)),
              pl.BlockSpec((1, _GF), lambda i: (0, 0)),
              pl.BlockSpec((2, 64), lambda i: (0, 0)),
              pl.BlockSpec((1, 64), lambda i: (0, 0)),
              pl.BlockSpec((64, _GF), lambda i: (0, 0)),
              pl.BlockSpec((1, _GF), lambda i: (0, 0))],
    out_specs=[pl.BlockSpec((_BE, _C), lambda i: (i, 0)),
               pl.BlockSpec((_BE, 8), lambda i: (i, 0))],
    out_shape=[jax.ShapeDtypeStruct((_E, _C), _f32),
               jax.ShapeDtypeStruct((_E, 8), _f32)],
)


def _node_update_body(h, chi, dh, dchi, wia, wca, wib, wcb, bia, bib, h_o, chi_o):
    inv = np.float32(1.0 / _AVG_NN)
    h2 = h[...] + (dh[0] + dh[1]) * inv
    chi2 = chi[...] + (dchi[0] + dchi[1]) * inv
    cn = _irrep_norm2(chi2)
    ya = _dot(h2, wia[...]) + _dot(cn, wca[...]) + bia[...]
    yb = _dot(h2, wib[...]) + _dot(cn, wcb[...]) + bib[...]
    h_o[...] = h2 + ya
    chi_o[...] = chi2 * (1.0 + _dot(yb, _rep28()))


_node_update = pl.pallas_call(
    _node_update_body,
    grid=(_GN,),
    in_specs=[pl.BlockSpec((_BN, _C), lambda i: (i, 0)),
              pl.BlockSpec((_BN, 8), lambda i: (i, 0)),
              pl.BlockSpec((2, _BN, _C), lambda i: (0, i, 0)),
              pl.BlockSpec((2, _BN, 8), lambda i: (0, i, 0)),
              pl.BlockSpec((_C, _C), lambda i: (0, 0)),
              pl.BlockSpec((2, _C), lambda i: (0, 0)),
              pl.BlockSpec((_C, 2), lambda i: (0, 0)),
              pl.BlockSpec((2, 2), lambda i: (0, 0)),
              pl.BlockSpec((1, _C), lambda i: (0, 0)),
              pl.BlockSpec((1, 2), lambda i: (0, 0))],
    out_specs=[pl.BlockSpec((_BN, _C), lambda i: (i, 0)),
               pl.BlockSpec((_BN, 8), lambda i: (i, 0))],
    out_shape=[jax.ShapeDtypeStruct((_N, _C), _f32),
               jax.ShapeDtypeStruct((_N, 8), _f32)],
)


def _readout_body(h, w0, b0, w1, b1, w2, b2, e_o):
    silu = jax.nn.silu
    e = silu(_dot(h[...], w0[...]) + b0[...])
    e = silu(_dot(e, w1[...]) + b1[...])
    e_o[...] = _dot(e, w2[...]) + b2[...]


_readout = pl.pallas_call(
    _readout_body,
    grid=(_GN,),
    in_specs=[pl.BlockSpec((_BN, _C), lambda i: (i, 0)),
              pl.BlockSpec((_C, _C), lambda i: (0, 0)),
              pl.BlockSpec((1, _C), lambda i: (0, 0)),
              pl.BlockSpec((_C, _C), lambda i: (0, 0)),
              pl.BlockSpec((1, _C), lambda i: (0, 0)),
              pl.BlockSpec((_C, 1), lambda i: (0, 0)),
              pl.BlockSpec((1, 1), lambda i: (0, 0))],
    out_specs=[pl.BlockSpec((_BN, 1), lambda i: (i, 0))],
    out_shape=[jax.ShapeDtypeStruct((_N, 1), _f32)],
)


_NC = 2
_NS = 16
_NW = _NC * _NS
_CH = 128
_NCHUNK = _E // _CH
_CPW = -(-_NCHUNK // _NW)
_NCH_HALF = _NCHUNK // _NC
_CPT = -(-_NCH_HALF // _NS)
_NROW = 624
_TAIL0 = _NROW * _NS
_TAILN = _N - _TAIL0

_sc_cache = {}


def _sc_mesh():
    return plsc.VectorSubcoreMesh(core_axis_name="c", subcore_axis_name="s",
                                  num_cores=_NC, num_subcores=_NS)


_sc_params = pltpu.CompilerParams(use_tc_tiling_on_sc=False)


def _gather_body(r_tab, s_tab, snd, rcv, r_out, s_out, ridx, sidx, rbuf, sbuf, sem1, sem2):
    wid = lax.axis_index("s") * _NC + lax.axis_index("c")

    def step(i, carry):
        c = wid + i * _NW

        @pl.when(c < _NCHUNK)
        def _():
            base = c * _CH
            pltpu.sync_copy(rcv.at[pl.ds(base, _CH)], ridx)
            pltpu.sync_copy(snd.at[pl.ds(base, _CH)], sidx)
            cp1 = pltpu.async_copy(r_tab.at[ridx], rbuf, sem1)
            cp2 = pltpu.async_copy(s_tab.at[sidx], sbuf, sem2)
            cp1.wait()
            cp2.wait()
            pltpu.sync_copy(rbuf, r_out.at[pl.ds(base, _CH)])
            pltpu.sync_copy(sbuf, s_out.at[pl.ds(base, _CH)])

        return carry

    lax.fori_loop(0, _CPW, step, 0)


def _gather(r_tab, s_tab, snd, rcv):
    if "gather" not in _sc_cache:
        _sc_cache["gather"] = functools.partial(
            pl.kernel,
            out_type=(jax.ShapeDtypeStruct((_E, _RD), _f32),
                      jax.ShapeDtypeStruct((_E, _SD), _f32)),
            mesh=_sc_mesh(),
            compiler_params=_sc_params,
            scratch_types=(pltpu.VMEM((_CH,), jnp.int32),
                           pltpu.VMEM((_CH,), jnp.int32),
                           pltpu.VMEM((_CH, _RD), _f32),
                           pltpu.VMEM((_CH, _SD), _f32),
                           pltpu.SemaphoreType.DMA,
                           pltpu.SemaphoreType.DMA),
        )(_gather_body)
    return _sc_cache["gather"](r_tab, s_tab, snd, rcv)


def _make_scatter(dims):
    out_type = tuple(jax.ShapeDtypeStruct((_NC, _N, d), _f32) for d in dims)
    scratch = [pltpu.VMEM((_CH,), jnp.int32)]
    for d in dims:
        scratch.append(pltpu.VMEM((_CH, d), _f32))
    for d in dims:
        scratch.append(pltpu.VMEM_SHARED((_N, d), _f32))

    @functools.partial(pl.kernel, out_type=out_type, mesh=_sc_mesh(),
                       compiler_params=_sc_params,
                       scratch_types=tuple(scratch))
    def _scatter(*refs):
        nvals = len(dims)
        vals = refs[:nvals]
        rcv = refs[nvals]
        zeros = refs[nvals + 1:2 * nvals + 1]
        outs = refs[2 * nvals + 1:3 * nvals + 1]
        idx = refs[3 * nvals + 1]
        bufs = refs[3 * nvals + 2:4 * nvals + 2]
        accs = refs[4 * nvals + 2:5 * nvals + 2]

        cid = lax.axis_index("c")
        sid = lax.axis_index("s")
        row0 = sid * _NROW

        for z, a in zip(zeros, accs):
            pltpu.sync_copy(z.at[pl.ds(row0, _NROW)], a.at[pl.ds(row0, _NROW)])

        @pl.when(sid == _NS - 1)
        def _():
            for z, a in zip(zeros, accs):
                pltpu.sync_copy(z.at[pl.ds(_TAIL0, _TAILN)],
                                a.at[pl.ds(_TAIL0, _TAILN)])

        plsc.subcore_barrier()

        def step(i, carry):
            lc = sid + i * _NS

            @pl.when(lc < _NCH_HALF)
            def _():
                base = (cid * _NCH_HALF + lc) * _CH
                pltpu.sync_copy(rcv.at[pl.ds(base, _CH)], idx)
                for v, b, a in zip(vals, bufs, accs):
                    pltpu.sync_copy(v.at[pl.ds(base, _CH)], b)
                    pltpu.sync_copy(b, a.at[idx], add=True)

            return carry

        lax.fori_loop(0, _CPT, step, 0)
        plsc.subcore_barrier()

        for a, o in zip(accs, outs):
            pltpu.sync_copy(a.at[pl.ds(row0, _NROW)],
                            o.at[cid, pl.ds(row0, _NROW)])

        @pl.when(sid == _NS - 1)
        def _():
            for a, o in zip(accs, outs):
                pltpu.sync_copy(a.at[pl.ds(_TAIL0, _TAILN)],
                                o.at[cid, pl.ds(_TAIL0, _TAILN)])

    return _scatter


def _scatter8(*args):
    if "s8" not in _sc_cache:
        _sc_cache["s8"] = _make_scatter([8])
    return _sc_cache["s8"](*args)


def _scatter_layer(*args):
    if "sl" not in _sc_cache:
        _sc_cache["sl"] = _make_scatter([_C, 8])
    return _sc_cache["sl"](*args)


def kernel(edge_vectors, distances, cutoffs, node_species, senders, receivers, params):
    d2 = distances.reshape(_E, 1)
    c2 = cutoffs.reshape(_E, 1)
    sp2 = node_species.reshape(_N, 1).astype(jnp.int32)
    snd = senders.astype(jnp.int32)
    rcv = receivers.astype(jnp.int32)

    ef, sh, shc = _edge_pre(edge_vectors, d2, c2)

    z8 = jnp.zeros((_N, 8), _f32)
    z128 = jnp.zeros((_N, _C), _f32)

    (chi_p,) = _scatter8(shc, rcv, z8)
    h, chi = _node_init(sp2, chi_p, chi_p, params['embed'])

    for lp in params['layers']:
        r_tab, s_tab = _pack_qk(h, chi, lp['Wq'], lp['Wk'], lp['Wqg'], lp['Wkg'])
        r_g, s_g = _gather(r_tab, s_tab, snd, rcv)
        msg, mchi = _edge_compute(
            ef, r_g, s_g, sh, c2,
            lp['fbr_W1'], lp['fbr_b1'].reshape(1, -1),
            lp['fbr_W2'], lp['fbr_b2'].reshape(1, -1),
            lp['fbs_W1'], lp['fbs_b1'].reshape(1, -1),
            lp['fbs_W2'], lp['fbs_b2'].reshape(1, -1),
            lp['gbr_W1'], lp['gbr_b1'].reshape(1, -1),
            lp['gbr_W2'], lp['gbr_b2'].reshape(1, -1),
            lp['gbs_W1'], lp['gbs_b1'].reshape(1, -1),
            lp['gbs_W2'], lp['gbs_b2'].reshape(1, -1))
        dh_p, dchi_p = _scatter_layer(msg, mchi, rcv, z128, z8)
        wi = lp['Wi']
        bi = lp['bi']
        h, chi = _node_update(
            h, chi, dh_p, dchi_p,
            wi[:_C, :_C], wi[_C:, :_C], wi[:_C, _C:], wi[_C:, _C:],
            bi[:_C].reshape(1, _C), bi[_C:].reshape(1, 2))

    (e,) = _readout(
        h, params['We0'], params['be0'].reshape(1, _C),
        params['We1'], params['be1'].reshape(1, _C),
        params['We2'], params['be2'].reshape(1, 1))
    return e.reshape(_N)


# speedup vs baseline: 12.1945x; 1.2542x over previous
import functools

import jax
import jax.numpy as jnp
import numpy as np
from jax import lax
from jax.experimental import pallas as pl
from jax.experimental.pallas import tpu as pltpu
from jax.experimental.pallas import tpu_sc as plsc

_N = 10000
_E = 320000
_C = 128
_H = 4
_DH = _C // _H
_DG = 16
_GF = 32
_NRBF = 32
_NSP = 100
_CUTOFF = 5.0
_AVG_NN = 32.0
_SPHC_NORM = 32.0

_BE = 2560
_GE = _E // _BE
_BN = 2000
_GN = _N // _BN

_RD = 256
_SD = 384

_f32 = jnp.float32


def _sel(nin, nout, group):
    d = lax.broadcasted_iota(jnp.int32, (nin, nout), 0) // group
    h = lax.broadcasted_iota(jnp.int32, (nin, nout), 1)
    return jnp.where(d == h, 1.0, 0.0).astype(_f32)


def _selT(nin, nout, group):
    h = lax.broadcasted_iota(jnp.int32, (nin, nout), 0)
    d = lax.broadcasted_iota(jnp.int32, (nin, nout), 1) // group
    return jnp.where(d == h, 1.0, 0.0).astype(_f32)


def _rep28():
    i = lax.broadcasted_iota(jnp.int32, (2, 8), 0)
    j = lax.broadcasted_iota(jnp.int32, (2, 8), 1)
    return jnp.where((j < 3) == (i == 0), 1.0, 0.0).astype(_f32)


def _irrep_norm2(a):
    return jnp.concatenate(
        [jnp.sum(a[:, :3] * a[:, :3], axis=-1, keepdims=True),
         jnp.sum(a[:, 3:] * a[:, 3:], axis=-1, keepdims=True)], axis=-1)


def _dot(a, b):
    return jnp.dot(a, b, preferred_element_type=_f32)


def _edge_pre_body(ev, d, cut, es_o):
    mu0 = np.float32(np.exp(-_CUTOFF))
    dmu = np.float32((1.0 - np.exp(-_CUTOFF)) / (_NRBF - 1))
    mus = mu0 + dmu * lax.broadcasted_iota(jnp.int32, (1, _NRBF), 1).astype(_f32)
    beta = np.float32((2.0 / _NRBF * (1.0 - np.exp(-_CUTOFF))) ** -2)
    d1 = d[...]
    ef = jnp.exp(-beta * (jnp.exp(-d1) - mus) ** 2)
    v = ev[...]
    r = jnp.sqrt(jnp.sum(v * v, axis=-1, keepdims=True) + 1e-12)
    u = v / r
    x = u[:, 0:1]
    y = u[:, 1:2]
    z = u[:, 2:3]
    s3 = np.float32(3.0 ** 0.5)
    s15 = np.float32(15.0 ** 0.5)
    s5 = np.float32(5.0 ** 0.5)
    sh = jnp.concatenate(
        [s3 * x, s3 * y, s3 * z,
         s15 * x * y, s15 * y * z, 0.5 * s5 * (3.0 * z * z - 1.0),
         s15 * x * z, 0.5 * s15 * (x * x - y * y)], axis=-1)
    cutv = cut[...]
    es_o[...] = jnp.concatenate(
        [ef, sh, sh * cutv, cutv,
         jnp.zeros((sh.shape[0], 79), _f32)], axis=-1)


_edge_pre = pl.pallas_call(
    _edge_pre_body,
    grid=(_GE,),
    in_specs=[pl.BlockSpec((_BE, 3), lambda i: (i, 0)),
              pl.BlockSpec((_BE, 1), lambda i: (i, 0)),
              pl.BlockSpec((_BE, 1), lambda i: (i, 0))],
    out_specs=[pl.BlockSpec((_BE, _C), lambda i: (i, 0))],
    out_shape=[jax.ShapeDtypeStruct((_E, _C), _f32)],
)


def _node_init_body(sp, chi0, chi1, emb, h_o, chi_o):
    s = sp[...]
    ids = lax.broadcasted_iota(jnp.int32, (1, _NSP), 1)
    oh = (s == ids).astype(_f32)
    h_o[...] = _dot(oh, emb[...])
    chi_o[...] = (chi0[0] + chi1[0]) * np.float32(1.0 / _SPHC_NORM)


_node_init = pl.pallas_call(
    _node_init_body,
    grid=(_GN,),
    in_specs=[pl.BlockSpec((_BN, 1), lambda i: (i, 0)),
              pl.BlockSpec((1, _BN, 8), lambda i: (0, i, 0)),
              pl.BlockSpec((1, _BN, 8), lambda i: (1, i, 0)),
              pl.BlockSpec((_NSP, _C), lambda i: (0, 0))],
    out_specs=[pl.BlockSpec((_BN, _C), lambda i: (i, 0)),
               pl.BlockSpec((_BN, 8), lambda i: (i, 0))],
    out_shape=[jax.ShapeDtypeStruct((_N, _C), _f32),
               jax.ShapeDtypeStruct((_N, 8), _f32)],
)


def _pack_qk_body(h, chi, wq, wk, wqg, wkg, r_o, s_o):
    hh = h[...]
    chiv = chi[...]
    z = jnp.zeros((hh.shape[0], 88), _f32)
    r_o[...] = jnp.concatenate(
        [_dot(hh, wq[...]), _dot(hh, wqg[...]), chiv, z], axis=-1)
    s_o[...] = jnp.concatenate(
        [_dot(hh, wk[...]), hh, _dot(hh, wkg[...]), chiv, z], axis=-1)


_pack_qk = pl.pallas_call(
    _pack_qk_body,
    grid=(_GN,),
    in_specs=[pl.BlockSpec((_BN, _C), lambda i: (i, 0)),
              pl.BlockSpec((_BN, 8), lambda i: (i, 0)),
              pl.BlockSpec((_C, _C), lambda i: (0, 0)),
              pl.BlockSpec((_C, _C), lambda i: (0, 0)),
              pl.BlockSpec((_C, _GF), lambda i: (0, 0)),
              pl.BlockSpec((_C, _GF), lambda i: (0, 0))],
    out_specs=[pl.BlockSpec((_BN, _RD), lambda i: (i, 0)),
               pl.BlockSpec((_BN, _SD), lambda i: (i, 0))],
    out_shape=[jax.ShapeDtypeStruct((_N, _RD), _f32),
               jax.ShapeDtypeStruct((_N, _SD), _f32)],
)


def _edge_compute_body(es, rg, sg,
                       fbrW1, fbrb1, fbrW2, fbrb2,
                       fbsW1, fbsb1, fbsW2, fbsb2,
                       gbrW1, gbrb1, gbrW2, gbrb2,
                       gbsW1, gbsb1, gbsW2, gbsb2,
                       msg_o, mchi_o):
    silu = jax.nn.silu
    esv = es[...]
    rgv = rg[...]
    sgv = sg[...]
    qr = rgv[:, :_C]
    qgr = rgv[:, _C:_C + _GF]
    chir = rgv[:, _C + _GF:_C + _GF + 8]
    ks = sgv[:, :_C]
    vs = sgv[:, _C:2 * _C]
    kgs = sgv[:, 2 * _C:2 * _C + _GF]
    chis = sgv[:, 2 * _C + _GF:2 * _C + _GF + 8]
    chij = chis - chir
    cs = _irrep_norm2(chij)
    efv = esv[:, :_NRBF]
    shv = esv[:, _NRBF:_NRBF + 8]
    cutv = esv[:, 48:49]
    w = (_dot(silu(_dot(efv, fbrW1[...]) + fbrb1[...]), fbrW2[...]) + fbrb2[...]
         + _dot(silu(_dot(cs, fbsW1[...]) + fbsb1[...]), fbsW2[...]) + fbsb2[...])
    p = qr * w * ks
    alpha = _dot(p, _sel(_C, _H, _DH)) * np.float32(_DH ** -0.5) * cutv
    msg_o[...] = _dot(alpha, _selT(_H, _C, _DH)) * vs
    wg = (_dot(silu(_dot(efv, gbrW1[...]) + gbrb1[...]), gbrW2[...]) + gbrb2[...]
          + _dot(silu(_dot(cs, gbsW1[...]) + gbsb1[...]), gbsW2[...]) + gbsb2[...])
    pg = qgr * wg * kgs
    ag = _dot(pg, _sel(_GF, 2, _DG)) * np.float32(_DG ** -0.5) * cutv
    mchi_o[...] = _dot(ag, _rep28()) * shv


_edge_compute = pl.pallas_call(
    _edge_compute_body,
    grid=(_GE,),
    in_specs=[pl.BlockSpec((_BE, _C), lambda i: (i, 0)),
              pl.BlockSpec((_BE, _RD), lambda i: (i, 0)),
              pl.BlockSpec((_BE, _SD), lambda i: (i, 0)),
              pl.BlockSpec((_NRBF, 128), lambda i: (0, 0)),
              pl.BlockSpec((1, 128), lambda i: (0, 0)),
              pl.BlockSpec((128, _C), lambda i: (0, 0)),
              pl.BlockSpec((1, _C), lambda i: (0, 0)),
              pl.BlockSpec((2, 128), lambda i: (0, 0)),
              pl.BlockSpec((1, 128), lambda i: (0, 0)),
              pl.BlockSpec((128, _C), lambda i: (0, 0)),
              pl.BlockSpec((1, _C), lambda i: (0, 0)),
              pl.BlockSpec((_NRBF, 64), lambda i: (0, 0)),
              pl.BlockSpec((1, 64), lambda i: (0, 0)),
              pl.BlockSpec((64, _GF), lambda i: (0, 0)),
              pl.BlockSpec((1, _GF), lambda i: (0, 0)),
              pl.BlockSpec((2, 64), lambda i: (0, 0)),
              pl.BlockSpec((1, 64), lambda i: (0, 0)),
              pl.BlockSpec((64, _GF), lambda i: (0, 0)),
              pl.BlockSpec((1, _GF), lambda i: (0, 0))],
    out_specs=[pl.BlockSpec((_BE, _C), lambda i: (i, 0)),
               pl.BlockSpec((_BE, 8), lambda i: (i, 0))],
    out_shape=[jax.ShapeDtypeStruct((_E, _C), _f32),
               jax.ShapeDtypeStruct((_E, 8), _f32)],
)


def _node_update_body(h, chi, dh, dchi, wia, wca, wib, wcb, bia, bib, h_o, chi_o):
    inv = np.float32(1.0 / _AVG_NN)
    h2 = h[...] + (dh[0] + dh[1]) * inv
    chi2 = chi[...] + (dchi[0] + dchi[1]) * inv
    cn = _irrep_norm2(chi2)
    ya = _dot(h2, wia[...]) + _dot(cn, wca[...]) + bia[...]
    yb = _dot(h2, wib[...]) + _dot(cn, wcb[...]) + bib[...]
    h_o[...] = h2 + ya
    chi_o[...] = chi2 * (1.0 + _dot(yb, _rep28()))


_node_update = pl.pallas_call(
    _node_update_body,
    grid=(_GN,),
    in_specs=[pl.BlockSpec((_BN, _C), lambda i: (i, 0)),
              pl.BlockSpec((_BN, 8), lambda i: (i, 0)),
              pl.BlockSpec((2, _BN, _C), lambda i: (0, i, 0)),
              pl.BlockSpec((2, _BN, 8), lambda i: (0, i, 0)),
              pl.BlockSpec((_C, _C), lambda i: (0, 0)),
              pl.BlockSpec((2, _C), lambda i: (0, 0)),
              pl.BlockSpec((_C, 2), lambda i: (0, 0)),
              pl.BlockSpec((2, 2), lambda i: (0, 0)),
              pl.BlockSpec((1, _C), lambda i: (0, 0)),
              pl.BlockSpec((1, 2), lambda i: (0, 0))],
    out_specs=[pl.BlockSpec((_BN, _C), lambda i: (i, 0)),
               pl.BlockSpec((_BN, 8), lambda i: (i, 0))],
    out_shape=[jax.ShapeDtypeStruct((_N, _C), _f32),
               jax.ShapeDtypeStruct((_N, 8), _f32)],
)


def _readout_body(h, w0, b0, w1, b1, w2, b2, e_o):
    silu = jax.nn.silu
    e = silu(_dot(h[...], w0[...]) + b0[...])
    e = silu(_dot(e, w1[...]) + b1[...])
    e_o[...] = _dot(e, w2[...]) + b2[...]


_readout = pl.pallas_call(
    _readout_body,
    grid=(_GN,),
    in_specs=[pl.BlockSpec((_BN, _C), lambda i: (i, 0)),
              pl.BlockSpec((_C, _C), lambda i: (0, 0)),
              pl.BlockSpec((1, _C), lambda i: (0, 0)),
              pl.BlockSpec((_C, _C), lambda i: (0, 0)),
              pl.BlockSpec((1, _C), lambda i: (0, 0)),
              pl.BlockSpec((_C, 1), lambda i: (0, 0)),
              pl.BlockSpec((1, 1), lambda i: (0, 0))],
    out_specs=[pl.BlockSpec((_BN, 1), lambda i: (i, 0))],
    out_shape=[jax.ShapeDtypeStruct((_N, 1), _f32)],
)


_NC = 2
_NS = 16
_NW = _NC * _NS
_CH = 128
_NCHUNK = _E // _CH
_CPW = -(-_NCHUNK // _NW)
_NCH_HALF = _NCHUNK // _NC
_CPT = -(-_NCH_HALF // _NS)
_NROW = 624
_TAIL0 = _NROW * _NS
_TAILN = _N - _TAIL0

_sc_cache = {}


def _sc_mesh():
    return plsc.VectorSubcoreMesh(core_axis_name="c", subcore_axis_name="s",
                                  num_cores=_NC, num_subcores=_NS)


_sc_params = pltpu.CompilerParams(use_tc_tiling_on_sc=False)


def _gather_body(r_tab, s_tab, snd, rcv, r_out, s_out, ridx, sidx, rbuf, sbuf, sem1, sem2):
    wid = lax.axis_index("s") * _NC + lax.axis_index("c")

    def step(i, carry):
        c = wid + i * _NW

        @pl.when(c < _NCHUNK)
        def _():
            base = c * _CH
            pltpu.sync_copy(rcv.at[pl.ds(base, _CH)], ridx)
            pltpu.sync_copy(snd.at[pl.ds(base, _CH)], sidx)
            cp1 = pltpu.async_copy(r_tab.at[ridx], rbuf, sem1)
            cp2 = pltpu.async_copy(s_tab.at[sidx], sbuf, sem2)
            cp1.wait()
            cp2.wait()
            pltpu.sync_copy(rbuf, r_out.at[pl.ds(base, _CH)])
            pltpu.sync_copy(sbuf, s_out.at[pl.ds(base, _CH)])

        return carry

    lax.fori_loop(0, _CPW, step, 0)


def _gather(r_tab, s_tab, snd, rcv):
    if "gather" not in _sc_cache:
        _sc_cache["gather"] = functools.partial(
            pl.kernel,
            out_type=(jax.ShapeDtypeStruct((_E, _RD), _f32),
                      jax.ShapeDtypeStruct((_E, _SD), _f32)),
            mesh=_sc_mesh(),
            scratch_types=(pltpu.VMEM((_CH,), jnp.int32),
                           pltpu.VMEM((_CH,), jnp.int32),
                           pltpu.VMEM((_CH, _RD), _f32),
                           pltpu.VMEM((_CH, _SD), _f32),
                           pltpu.SemaphoreType.DMA,
                           pltpu.SemaphoreType.DMA),
        )(_gather_body)
    return _sc_cache["gather"](r_tab, s_tab, snd, rcv)


def _make_scatter(dims):
    out_type = tuple(jax.ShapeDtypeStruct((_NC, _N, d), _f32) for d in dims)
    scratch = [pltpu.VMEM((_CH,), jnp.int32)]
    for d in dims:
        scratch.append(pltpu.VMEM((_CH, d), _f32))
    for d in dims:
        scratch.append(pltpu.VMEM_SHARED((_N, d), _f32))

    @functools.partial(pl.kernel, out_type=out_type, mesh=_sc_mesh(),
                       compiler_params=_sc_params,
                       scratch_types=tuple(scratch))
    def _scatter(*refs):
        nvals = len(dims)
        vals = refs[:nvals]
        rcv = refs[nvals]
        zeros = refs[nvals + 1:2 * nvals + 1]
        outs = refs[2 * nvals + 1:3 * nvals + 1]
        idx = refs[3 * nvals + 1]
        bufs = refs[3 * nvals + 2:4 * nvals + 2]
        accs = refs[4 * nvals + 2:5 * nvals + 2]

        cid = lax.axis_index("c")
        sid = lax.axis_index("s")
        row0 = sid * _NROW

        for z, a in zip(zeros, accs):
            pltpu.sync_copy(z.at[pl.ds(row0, _NROW)], a.at[pl.ds(row0, _NROW)])

        @pl.when(sid == _NS - 1)
        def _():
            for z, a in zip(zeros, accs):
                pltpu.sync_copy(z.at[pl.ds(_TAIL0, _TAILN)],
                                a.at[pl.ds(_TAIL0, _TAILN)])

        plsc.subcore_barrier()

        def step(i, carry):
            lc = sid + i * _NS

            @pl.when(lc < _NCH_HALF)
            def _():
                base = (cid * _NCH_HALF + lc) * _CH
                pltpu.sync_copy(rcv.at[pl.ds(base, _CH)], idx)
                for v, b, a in zip(vals, bufs, accs):
                    pltpu.sync_copy(v.at[pl.ds(base, _CH)], b)
                    pltpu.sync_copy(b, a.at[idx], add=True)

            return carry

        lax.fori_loop(0, _CPT, step, 0)
        plsc.subcore_barrier()

        for a, o in zip(accs, outs):
            pltpu.sync_copy(a.at[pl.ds(row0, _NROW)],
                            o.at[cid, pl.ds(row0, _NROW)])

        @pl.when(sid == _NS - 1)
        def _():
            for a, o in zip(accs, outs):
                pltpu.sync_copy(a.at[pl.ds(_TAIL0, _TAILN)],
                                o.at[cid, pl.ds(_TAIL0, _TAILN)])

    return _scatter


def _scatter8(*args):
    if "s8" not in _sc_cache:
        _sc_cache["s8"] = _make_scatter([8])
    return _sc_cache["s8"](*args)


def _scatter_layer(*args):
    if "sl" not in _sc_cache:
        _sc_cache["sl"] = _make_scatter([_C, 8])
    return _sc_cache["sl"](*args)


def kernel(edge_vectors, distances, cutoffs, node_species, senders, receivers, params):
    d2 = distances.reshape(_E, 1)
    c2 = cutoffs.reshape(_E, 1)
    sp2 = node_species.reshape(_N, 1).astype(jnp.int32)
    snd = senders.astype(jnp.int32)
    rcv = receivers.astype(jnp.int32)

    (es,) = _edge_pre(edge_vectors, d2, c2)
    shc = es[:, 40:48]

    z8 = jnp.zeros((_N, 8), _f32)
    z128 = jnp.zeros((_N, _C), _f32)

    (chi_p,) = _scatter8(shc, rcv, z8)
    h, chi = _node_init(sp2, chi_p, chi_p, params['embed'])

    for lp in params['layers']:
        r_tab, s_tab = _pack_qk(h, chi, lp['Wq'], lp['Wk'], lp['Wqg'], lp['Wkg'])
        r_g, s_g = _gather(r_tab, s_tab, snd, rcv)
        msg, mchi = _edge_compute(
            es, r_g, s_g,
            lp['fbr_W1'], lp['fbr_b1'].reshape(1, -1),
            lp['fbr_W2'], lp['fbr_b2'].reshape(1, -1),
            lp['fbs_W1'], lp['fbs_b1'].reshape(1, -1),
            lp['fbs_W2'], lp['fbs_b2'].reshape(1, -1),
            lp['gbr_W1'], lp['gbr_b1'].reshape(1, -1),
            lp['gbr_W2'], lp['gbr_b2'].reshape(1, -1),
            lp['gbs_W1'], lp['gbs_b1'].reshape(1, -1),
            lp['gbs_W2'], lp['gbs_b2'].reshape(1, -1))
        dh_p, dchi_p = _scatter_layer(msg, mchi, rcv, z128, z8)
        wi = lp['Wi']
        bi = lp['bi']
        h, chi = _node_update(
            h, chi, dh_p, dchi_p,
            wi[:_C, :_C], wi[_C:, :_C], wi[:_C, _C:], wi[_C:, _C:],
            bi[:_C].reshape(1, _C), bi[_C:].reshape(1, 2))

    (e,) = _readout(
        h, params['We0'], params['be0'].reshape(1, _C),
        params['We1'], params['be1'].reshape(1, _C),
        params['We2'], params['be2'].reshape(1, 1))
    return e.reshape(_N)
